# baseline (speedup 1.0000x reference)
import functools

import jax
import jax.numpy as jnp
from jax import lax
from jax.experimental import pallas as pl
from jax.experimental.pallas import tpu as pltpu
from jax.experimental.pallas import tpu_sc as plsc


def _sc_call(x_flat, rt_flat, *, S, H, max_len):
    NW = 32
    JW = S // NW
    CI = 8
    NCH = S // CI
    XB = JW * H
    W = (CI + JW - 1) * H
    row_bytes = S * H

    mesh = plsc.VectorSubcoreMesh(core_axis_name="c", subcore_axis_name="s")

    @functools.partial(
        pl.kernel,
        mesh=mesh,
        out_type=jax.ShapeDtypeStruct((S, S * H), jnp.float32),
        scratch_types=[
            pltpu.VMEM((XB,), jnp.float32),
            pltpu.VMEM((W,), jnp.float32),
            pltpu.VMEM((W,), jnp.float32),
            pltpu.VMEM((CI, XB), jnp.float32),
            pltpu.VMEM((CI, XB), jnp.float32),
            pltpu.SemaphoreType.DMA,
            pltpu.SemaphoreType.DMA,
            pltpu.SemaphoreType.DMA,
            pltpu.SemaphoreType.DMA,
        ],
    )
    def k(x_hbm, rt_hbm, out_hbm, xbuf, rta, rtb, outa, outb,
          sla, slb, soa, sob):
        wid = lax.axis_index("s") * 2 + lax.axis_index("c")
        j0 = wid * JW

        def rt_src(c):
            start = (max_len - c * CI - (CI - 1) + j0) * H
            return rt_hbm.at[pl.ds(start, W)]

        def out_dst(c):
            return out_hbm.at[pl.ds(c * CI, CI), pl.ds(j0 * H, XB)]

        def compute(rtbuf, outbuf):
            @plsc.parallel_loop(0, XB, 16, unroll=2)
            def _inner(b):
                xv = xbuf[pl.ds(b, 16)]
                for r in range(CI):
                    outbuf[r, pl.ds(b, 16)] = (
                        xv + rtbuf[pl.ds((CI - 1 - r) * H + b, 16)]
                    )

        pltpu.sync_copy(x_hbm.at[pl.ds(j0 * H, XB)], xbuf)
        pltpu.make_async_copy(rt_src(1), rtb, slb).start()
        pltpu.sync_copy(rt_src(0), rta)
        compute(rta, outa)
        pltpu.make_async_copy(outa, out_dst(0), soa).start()
        pltpu.make_async_copy(rt_src(2), rta, sla).start()
        pltpu.make_async_copy(rt_src(1), rtb, slb).wait()
        compute(rtb, outb)
        pltpu.make_async_copy(outb, out_dst(1), sob).start()

        def pair(cp, carry):
            ca = 2 * cp
            cb = 2 * cp + 1
            ca_next = jnp.minimum(ca + 2, NCH - 1)
            pltpu.make_async_copy(rt_src(cb), rtb, slb).start()
            pltpu.make_async_copy(rt_src(ca), rta, sla).wait()
            pltpu.make_async_copy(outa, out_dst(0), soa).wait()
            compute(rta, outa)
            pltpu.make_async_copy(outa, out_dst(ca), soa).start()
            pltpu.make_async_copy(rt_src(ca_next), rta, sla).start()
            pltpu.make_async_copy(rt_src(cb), rtb, slb).wait()
            pltpu.make_async_copy(outb, out_dst(1), sob).wait()
            compute(rtb, outb)
            pltpu.make_async_copy(outb, out_dst(cb), sob).start()
            return carry

        lax.fori_loop(1, NCH // 2, pair, 0)

        pltpu.make_async_copy(rt_src(NCH - 1), rta, sla).wait()
        pltpu.make_async_copy(outa, out_dst(0), soa).wait()
        pltpu.make_async_copy(outb, out_dst(1), sob).wait()

    return k(x_flat, rt_flat)


def kernel(x, rel_pos_embeddings):
    _, S, H = x.shape
    n_rows = rel_pos_embeddings.shape[0]
    max_len = (n_rows - 1) // 2
    pad = (-n_rows) % 8
    rt = jnp.pad(jnp.flip(rel_pos_embeddings, axis=0), ((0, pad), (0, 0)))
    out = _sc_call(
        x.reshape(S * H), rt.reshape(-1), S=S, H=H, max_len=max_len
    )
    return out.reshape(S, S, H)


# speedup vs baseline: 2.4522x; 2.4522x over previous
import functools

import jax
import jax.numpy as jnp
from jax import lax
from jax.experimental import pallas as pl
from jax.experimental.pallas import tpu as pltpu
from jax.experimental.pallas import tpu_sc as plsc


def _sc_call(x_flat, rt_flat, *, S, H, max_len):
    NW = 32
    JW = S // NW
    CI = 8
    NCH = S // CI
    XB = JW * H
    W = (CI + JW - 1) * H
    ROW = S * H

    mesh = plsc.VectorSubcoreMesh(core_axis_name="c", subcore_axis_name="s")

    @functools.partial(
        pl.kernel,
        mesh=mesh,
        out_type=jax.ShapeDtypeStruct((S * S * H,), jnp.float32),
        scratch_types=[
            pltpu.VMEM((XB,), jnp.float32),
            pltpu.VMEM((W,), jnp.float32),
            pltpu.VMEM((W,), jnp.float32),
            pltpu.VMEM((CI * XB,), jnp.float32),
            pltpu.VMEM((CI * XB,), jnp.float32),
            pltpu.SemaphoreType.DMA,
            pltpu.SemaphoreType.DMA,
            pltpu.SemaphoreType.DMA,
            pltpu.SemaphoreType.DMA,
        ],
    )
    def k(x_hbm, rt_hbm, out_hbm, xbuf, rta, rtb, outa, outb,
          sla, slb, soa, sob):
        wid = lax.axis_index("s") * 2 + lax.axis_index("c")
        j0 = wid * JW

        def rt_src(c):
            start = (max_len - c * CI - (CI - 1) + j0) * H
            return rt_hbm.at[pl.ds(start, W)]

        def compute(rtbuf, outbuf):
            @plsc.parallel_loop(0, XB, 16, unroll=2)
            def _inner(b):
                xv = xbuf[pl.ds(b, 16)]
                for r in range(CI):
                    outbuf[pl.ds(r * XB + b, 16)] = (
                        xv + rtbuf[pl.ds((CI - 1 - r) * H + b, 16)]
                    )

        def out_start(outbuf, sem, c):
            for r in range(CI):
                pltpu.make_async_copy(
                    outbuf.at[pl.ds(r * XB, XB)],
                    out_hbm.at[pl.ds((c * CI + r) * ROW + j0 * H, XB)],
                    sem,
                ).start()

        def out_wait(outbuf, sem):
            for r in range(CI):
                pltpu.make_async_copy(
                    outbuf.at[pl.ds(r * XB, XB)],
                    out_hbm.at[pl.ds(r * XB, XB)],
                    sem,
                ).wait()

        pltpu.sync_copy(x_hbm.at[pl.ds(j0 * H, XB)], xbuf)
        pltpu.make_async_copy(rt_src(1), rtb, slb).start()
        pltpu.sync_copy(rt_src(0), rta)
        compute(rta, outa)
        out_start(outa, soa, 0)
        pltpu.make_async_copy(rt_src(2), rta, sla).start()
        pltpu.make_async_copy(rt_src(1), rtb, slb).wait()
        compute(rtb, outb)
        out_start(outb, sob, 1)

        def pair(cp, carry):
            ca = 2 * cp
            cb = 2 * cp + 1
            ca_next = jnp.minimum(ca + 2, NCH - 1)
            pltpu.make_async_copy(rt_src(cb), rtb, slb).start()
            pltpu.make_async_copy(rt_src(ca), rta, sla).wait()
            out_wait(outa, soa)
            compute(rta, outa)
            out_start(outa, soa, ca)
            pltpu.make_async_copy(rt_src(ca_next), rta, sla).start()
            pltpu.make_async_copy(rt_src(cb), rtb, slb).wait()
            out_wait(outb, sob)
            compute(rtb, outb)
            out_start(outb, sob, cb)
            return carry

        lax.fori_loop(1, NCH // 2, pair, 0)

        pltpu.make_async_copy(rt_src(NCH - 1), rta, sla).wait()
        out_wait(outa, soa)
        out_wait(outb, sob)

    return k(x_flat, rt_flat)


def kernel(x, rel_pos_embeddings):
    _, S, H = x.shape
    n_rows = rel_pos_embeddings.shape[0]
    max_len = (n_rows - 1) // 2
    pad = (-n_rows) % 8
    rt = jnp.pad(jnp.flip(rel_pos_embeddings, axis=0), ((0, pad), (0, 0)))
    out = _sc_call(
        x.reshape(S * H), rt.reshape(-1), S=S, H=H, max_len=max_len
    )
    return out.reshape(S, S, H)
